# Initial kernel scaffold; baseline (speedup 1.0000x reference)
import jax
import jax.numpy as jnp
from jax.experimental import pallas as pl


def kernel(x, edge_index, edge_weight, W0, b0, g0, be0, W1, b1, g1, be1, W2, b2):
    raise NotImplementedError("write your pallas kernel here")


# speedup vs baseline: 26.0148x; 26.0148x over previous
import jax
import jax.numpy as jnp
from jax import lax
from jax.experimental import pallas as pl
from jax.experimental.pallas import tpu as pltpu
from jax.experimental.pallas import tpu_sc as plsc

N = 10000
NPAD = 10240
K = 80
NSC = 2
NTEC = 16
L = 16
SEG = NPAD // NTEC
BN_INV = 0.9999950000374997


def _mesh():
    return plsc.VectorSubcoreMesh(core_axis_name="c", subcore_axis_name="s")


def _zero_ref(ref, n):
    z = jnp.zeros((L,), jnp.float32)

    def body(i, _):
        ref[pl.ds(i * L, L)] = z
        return 0

    lax.fori_loop(0, n // L, body, 0)


def _newton_rsqrt(x):
    bits = plsc.bitcast(x, jnp.int32)
    y = plsc.bitcast(jnp.int32(0x5F3759DF) - jnp.right_shift(bits, 1),
                     jnp.float32)
    for _ in range(3):
        y = y * (1.5 - 0.5 * x * y * y)
    return y


def _norm_body(src3, dst3, ew3, norm_out, selfw_out,
               sslab, dslab, wslab, degacc, sumbuf, tmpbuf, disbuf, swbuf,
               dis_t, part_sh, dis_sh):
    c = lax.axis_index("c")
    s = lax.axis_index("s")
    rows = src3.shape[1]

    @pl.when(c == 0)
    def _():
        pltpu.sync_copy(src3.at[s], sslab)
        pltpu.sync_copy(dst3.at[s], dslab)
        pltpu.sync_copy(ew3.at[s], wslab)
        _zero_ref(degacc, NPAD)

        def deg_row(g, _):
            for k in range(K // L):
                d16 = dslab[g, pl.ds(k * L, L)]
                w16 = wslab[g, pl.ds(k * L, L)]
                plsc.addupdate_scatter(degacc, [d16], w16)
            return 0

        lax.fori_loop(0, rows, deg_row, 0)

        pltpu.sync_copy(degacc, part_sh.at[s])
        plsc.subcore_barrier()
        base = s * SEG
        pltpu.sync_copy(part_sh.at[0, pl.ds(base, SEG)], sumbuf)
        for j in range(1, NTEC):
            pltpu.sync_copy(part_sh.at[j, pl.ds(base, SEG)], tmpbuf)

            def acc_i(i, _):
                sl = pl.ds(i * L, L)
                sumbuf[sl] = sumbuf[sl] + tmpbuf[sl]
                return 0

            lax.fori_loop(0, SEG // L, acc_i, 0)

        def fin_i(i, _):
            sl = pl.ds(i * L, L)
            deg = sumbuf[sl] + 1.0
            dis = _newton_rsqrt(deg)
            disbuf[sl] = dis
            swbuf[sl] = dis * dis
            return 0

        lax.fori_loop(0, SEG // L, fin_i, 0)
        pltpu.sync_copy(disbuf, dis_sh.at[pl.ds(base, SEG)])
        pltpu.sync_copy(swbuf, selfw_out.at[pl.ds(base, SEG)])
        plsc.subcore_barrier()

        pltpu.sync_copy(dis_sh, dis_t)

        def norm_row(g, _):
            for k in range(K // L):
                sl = pl.ds(k * L, L)
                s16 = sslab[g, sl]
                d16 = dslab[g, sl]
                w16 = wslab[g, sl]
                dsv = plsc.load_gather(dis_t, [s16])
                ddv = plsc.load_gather(dis_t, [d16])
                wslab[g, sl] = dsv * w16 * ddv
            return 0

        lax.fori_loop(0, rows, norm_row, 0)
        pltpu.sync_copy(wslab, norm_out.at[s])


def _norm_kernel(src3, dst3, ew3):
    rows = src3.shape[1]
    return pl.kernel(
        _norm_body,
        out_type=(jax.ShapeDtypeStruct((NTEC, rows, K), jnp.float32),
                  jax.ShapeDtypeStruct((NPAD,), jnp.float32)),
        mesh=_mesh(),
        compiler_params=pltpu.CompilerParams(needs_layout_passes=False, use_tc_tiling_on_sc=False),
        scratch_types=[
            pltpu.VMEM((rows, K), jnp.int32),
            pltpu.VMEM((rows, K), jnp.int32),
            pltpu.VMEM((rows, K), jnp.float32),
            pltpu.VMEM((NPAD,), jnp.float32),
            pltpu.VMEM((SEG,), jnp.float32),
            pltpu.VMEM((SEG,), jnp.float32),
            pltpu.VMEM((SEG,), jnp.float32),
            pltpu.VMEM((SEG,), jnp.float32),
            pltpu.VMEM((NPAD,), jnp.float32),
            pltpu.VMEM_SHARED((NTEC, NPAD), jnp.float32),
            pltpu.VMEM_SHARED((NPAD,), jnp.float32),
        ],
    )(src3, dst3, ew3)


def _spmm_chunks(table, sslab, dslab, nslab, rows0, rows1, sem, acc_sh, nch):
    cw = table.shape[1]
    nvr = cw // L

    def start_gather(g, buf, sm):
        pltpu.async_copy(table.at[sslab.at[g]], buf, sm)

    def wait_gather(g, buf, sm):
        pltpu.make_async_copy(table.at[sslab.at[g]], buf, sm).wait()

    start_gather(0, rows0, sem.at[0])

    def chunk(g, _):
        b = lax.rem(g, 2)

        @pl.when(g + 1 < nch)
        def _():
            @pl.when(b == 0)
            def _():
                start_gather(g + 1, rows1, sem.at[1])

            @pl.when(b == 1)
            def _():
                start_gather(g + 1, rows0, sem.at[0])

        def scale_and_scatter(buf, sm):
            wait_gather(g, buf, sm)

            def grp(k, _):
                n16 = nslab[g, pl.ds(k * L, L)]
                for jj in range(L):
                    nv = jnp.broadcast_to(n16[jj], (L,))
                    e = k * L + jj
                    for v in range(nvr):
                        sl = pl.ds(v * L, L)
                        buf[e, sl] = buf[e, sl] * nv
                return 0

            lax.fori_loop(0, K // L, grp, 0)
            pltpu.sync_copy(buf, acc_sh.at[dslab.at[g]], add=True)

        @pl.when(b == 0)
        def _():
            scale_and_scatter(rows0, sem.at[0])

        @pl.when(b == 1)
        def _():
            scale_and_scatter(rows1, sem.at[1])

        return 0

    lax.fori_loop(0, nch, chunk, 0)


def _stage_slabs(src3, dst3, norm3, gidx, sslab, dslab, nslab):
    pltpu.sync_copy(src3.at[gidx], sslab)
    pltpu.sync_copy(dst3.at[gidx], dslab)
    pltpu.sync_copy(norm3.at[gidx], nslab)


def _zero_rows_buf(buf):
    cw = buf.shape[1]

    def zrow(i, _):
        for v in range(cw // L):
            buf[i, pl.ds(v * L, L)] = jnp.zeros((L,), jnp.float32)
        return 0

    lax.fori_loop(0, buf.shape[0], zrow, 0)


def _acc_zero(zsrc, acc_sh, s):
    base = s * (N // NTEC)
    for j in range(7):
        pltpu.sync_copy(zsrc, acc_sh.at[pl.ds(base + j * 80, 80)])
    pltpu.sync_copy(zsrc.at[pl.ds(0, 65)], acc_sh.at[pl.ds(base + 560, 65)])


def _acc_copyout(acc_sh, out_slice, s):
    base = s * (N // NTEC)
    for j in range(7):
        pltpu.sync_copy(acc_sh.at[pl.ds(base + j * 80, 80)],
                        out_slice.at[pl.ds(base + j * 80, 80)])
    pltpu.sync_copy(acc_sh.at[pl.ds(base + 560, 65)],
                    out_slice.at[pl.ds(base + 560, 65)])


def _spmm0_body(xt, src3, dst3, norm3, out,
                sslab, dslab, nslab, rows0, rows1, sem, acc_sh):
    c = lax.axis_index("c")
    s = lax.axis_index("s")
    _zero_rows_buf(rows0)
    _acc_zero(rows0, acc_sh, s)
    plsc.subcore_barrier()
    _stage_slabs(src3, dst3, norm3, c * NTEC + s, sslab, dslab, nslab)
    _spmm_chunks(xt, sslab, dslab, nslab, rows0, rows1, sem, acc_sh,
                 src3.shape[1])
    plsc.subcore_barrier()
    _acc_copyout(acc_sh, out.at[c], s)


def _spmm0_kernel(x, src3, dst3, norm3):
    cw = x.shape[1]
    rows = src3.shape[1]
    return pl.kernel(
        _spmm0_body,
        out_type=jax.ShapeDtypeStruct((NSC, N, cw), jnp.float32),
        mesh=_mesh(),
        compiler_params=pltpu.CompilerParams(needs_layout_passes=False, use_tc_tiling_on_sc=False),
        scratch_types=[
            pltpu.VMEM((rows, K), jnp.int32),
            pltpu.VMEM((rows, K), jnp.int32),
            pltpu.VMEM((rows, K), jnp.float32),
            pltpu.VMEM((K, cw), jnp.float32),
            pltpu.VMEM((K, cw), jnp.float32),
            pltpu.SemaphoreType.DMA((2,)),
            pltpu.VMEM_SHARED((N, cw), jnp.float32),
        ],
    )(x, src3, dst3, norm3)


def _spmm1_body(ha, hb, src3, dst3, norm3, out,
                sslab, dslab, nslab, rows0, rows1, sem, acc_sh):
    c = lax.axis_index("c")
    s = lax.axis_index("s")
    _zero_rows_buf(rows0)
    _acc_zero(rows0, acc_sh, s)
    plsc.subcore_barrier()
    nch = src3.shape[1]

    def run(table):
        for r in range(2):
            _stage_slabs(src3, dst3, norm3, 2 * s + r, sslab, dslab, nslab)
            _spmm_chunks(table, sslab, dslab, nslab, rows0, rows1, sem,
                         acc_sh, nch)

    @pl.when(c == 0)
    def _():
        run(ha)

    @pl.when(c == 1)
    def _():
        run(hb)

    plsc.subcore_barrier()
    _acc_copyout(acc_sh, out.at[c], s)


def _spmm1_kernel(ha, hb, src3, dst3, norm3):
    cw = ha.shape[1]
    rows = src3.shape[1]
    return pl.kernel(
        _spmm1_body,
        out_type=jax.ShapeDtypeStruct((NSC, N, cw), jnp.float32),
        mesh=_mesh(),
        compiler_params=pltpu.CompilerParams(needs_layout_passes=False, use_tc_tiling_on_sc=False),
        scratch_types=[
            pltpu.VMEM((rows, K), jnp.int32),
            pltpu.VMEM((rows, K), jnp.int32),
            pltpu.VMEM((rows, K), jnp.float32),
            pltpu.VMEM((K, cw), jnp.float32),
            pltpu.VMEM((K, cw), jnp.float32),
            pltpu.SemaphoreType.DMA((2,)),
            pltpu.VMEM_SHARED((N, cw), jnp.float32),
        ],
    )(ha, hb, src3, dst3, norm3)


def _spmm_z_body(z, src3, dst3, norm3, selfw, b2b, out,
                 sslab, dslab, nslab, z_t, acc_t, sumbuf, tmpbuf, b2v,
                 part_sh):
    c = lax.axis_index("c")
    s = lax.axis_index("s")
    rows = src3.shape[1]

    @pl.when(c == 0)
    def _():
        pltpu.sync_copy(src3.at[s], sslab)
        pltpu.sync_copy(dst3.at[s], dslab)
        pltpu.sync_copy(norm3.at[s], nslab)
        pltpu.sync_copy(z, z_t)
        _zero_ref(acc_t, NPAD)

        def row(g, _):
            for k in range(K // L):
                sl = pl.ds(k * L, L)
                s16 = sslab[g, sl]
                d16 = dslab[g, sl]
                n16 = nslab[g, sl]
                zv = plsc.load_gather(z_t, [s16])
                plsc.addupdate_scatter(acc_t, [d16], zv * n16)
            return 0

        lax.fori_loop(0, rows, row, 0)

        pltpu.sync_copy(acc_t, part_sh.at[s])
        plsc.subcore_barrier()
        base = s * SEG
        pltpu.sync_copy(part_sh.at[0, pl.ds(base, SEG)], sumbuf)
        for j in range(1, NTEC):
            pltpu.sync_copy(part_sh.at[j, pl.ds(base, SEG)], tmpbuf)

            def acc_i(i, _):
                sl = pl.ds(i * L, L)
                sumbuf[sl] = sumbuf[sl] + tmpbuf[sl]
                return 0

            lax.fori_loop(0, SEG // L, acc_i, 0)

        pltpu.sync_copy(selfw.at[pl.ds(base, SEG)], tmpbuf)
        pltpu.sync_copy(b2b, b2v)
        bv = b2v[...]

        def fin_i(i, _):
            sl = pl.ds(i * L, L)
            sumbuf[sl] = (sumbuf[sl] + tmpbuf[sl] * z_t[pl.ds(base + i * L, L)]
                          + bv)
            return 0

        lax.fori_loop(0, SEG // L, fin_i, 0)
        pltpu.sync_copy(sumbuf, out.at[pl.ds(base, SEG)])


def _spmm_z_kernel(z, src3, dst3, norm3, selfw, b2b):
    rows = src3.shape[1]
    return pl.kernel(
        _spmm_z_body,
        out_type=jax.ShapeDtypeStruct((NPAD,), jnp.float32),
        mesh=_mesh(),
        compiler_params=pltpu.CompilerParams(needs_layout_passes=False, use_tc_tiling_on_sc=False),
        scratch_types=[
            pltpu.VMEM((rows, K), jnp.int32),
            pltpu.VMEM((rows, K), jnp.int32),
            pltpu.VMEM((rows, K), jnp.float32),
            pltpu.VMEM((NPAD,), jnp.float32),
            pltpu.VMEM((NPAD,), jnp.float32),
            pltpu.VMEM((SEG,), jnp.float32),
            pltpu.VMEM((SEG,), jnp.float32),
            pltpu.VMEM((L,), jnp.float32),
            pltpu.VMEM_SHARED((NTEC, NPAD), jnp.float32),
        ],
    )(z, src3, dst3, norm3, selfw, b2b)


def _dense0_tc(y0p_ref, x_ref, sw_ref, w_ref, b_ref, g_ref, be_ref,
               oa_ref, ob_ref):
    y0 = y0p_ref[0] + y0p_ref[1] + sw_ref[...] * x_ref[...]
    h = jax.lax.dot_general(y0, w_ref[...], (((1,), (0,)), ((), ())),
                            precision=lax.Precision.HIGHEST,
                            preferred_element_type=jnp.float32)
    h = (h + b_ref[...]) * (g_ref[...] * BN_INV) + be_ref[...]
    h = jnp.maximum(h, 0.0)
    oa_ref[...] = h[:, :128]
    ob_ref[...] = h[:, 128:]


def _dense0(y0p, x, sw, W0, b0, g0, be0):
    blk = 1000
    grid = (N // blk,)
    return pl.pallas_call(
        _dense0_tc,
        grid=grid,
        in_specs=[
            pl.BlockSpec((NSC, blk, 128), lambda i: (0, i, 0)),
            pl.BlockSpec((blk, 128), lambda i: (i, 0)),
            pl.BlockSpec((blk, 1), lambda i: (i, 0)),
            pl.BlockSpec((128, 256), lambda i: (0, 0)),
            pl.BlockSpec((1, 256), lambda i: (0, 0)),
            pl.BlockSpec((1, 256), lambda i: (0, 0)),
            pl.BlockSpec((1, 256), lambda i: (0, 0)),
        ],
        out_specs=[
            pl.BlockSpec((blk, 128), lambda i: (i, 0)),
            pl.BlockSpec((blk, 128), lambda i: (i, 0)),
        ],
        out_shape=[jax.ShapeDtypeStruct((N, 128), jnp.float32),
                   jax.ShapeDtypeStruct((N, 128), jnp.float32)],
    )(y0p, x, sw, W0, b0, g0, be0)


def _dense1_tc(y1c_ref, ha_ref, hb_ref, sw_ref, w_ref, b_ref, g_ref, be_ref,
               w2_ref, z_ref):
    sw = sw_ref[...]
    hin = jnp.concatenate([y1c_ref[0] + sw * ha_ref[...],
                           y1c_ref[1] + sw * hb_ref[...]], axis=1)
    h = jax.lax.dot_general(hin, w_ref[...], (((1,), (0,)), ((), ())),
                            precision=lax.Precision.HIGHEST,
                            preferred_element_type=jnp.float32)
    h = (h + b_ref[...]) * (g_ref[...] * BN_INV) + be_ref[...]
    h = jnp.maximum(h, 0.0)
    z_ref[...] = jnp.sum(h * w2_ref[...], axis=1, keepdims=True)


def _dense1(y1c, ha, hb, sw, W1, b1, g1, be1, w2r):
    blk = 1000
    grid = (N // blk,)
    return pl.pallas_call(
        _dense1_tc,
        grid=grid,
        in_specs=[
            pl.BlockSpec((NSC, blk, 128), lambda i: (0, i, 0)),
            pl.BlockSpec((blk, 128), lambda i: (i, 0)),
            pl.BlockSpec((blk, 128), lambda i: (i, 0)),
            pl.BlockSpec((blk, 1), lambda i: (i, 0)),
            pl.BlockSpec((256, 256), lambda i: (0, 0)),
            pl.BlockSpec((1, 256), lambda i: (0, 0)),
            pl.BlockSpec((1, 256), lambda i: (0, 0)),
            pl.BlockSpec((1, 256), lambda i: (0, 0)),
            pl.BlockSpec((1, 256), lambda i: (0, 0)),
        ],
        out_specs=[pl.BlockSpec((blk, 1), lambda i: (i, 0))],
        out_shape=[jax.ShapeDtypeStruct((N, 1), jnp.float32)],
    )(y1c, ha, hb, sw, W1, b1, g1, be1, w2r)


def kernel(x, edge_index, edge_weight, W0, b0, g0, be0, W1, b1, g1, be1,
           W2, b2):
    E = edge_index.shape[1]
    assert E % (NSC * NTEC * K) == 0
    rows16 = E // K // NTEC
    rows32 = E // K // (NSC * NTEC)
    src3 = edge_index[0].reshape(NTEC, rows16, K)
    dst3 = edge_index[1].reshape(NTEC, rows16, K)
    ew3 = edge_weight.reshape(NTEC, rows16, K)

    norm3, selfw = _norm_kernel(src3, dst3, ew3)
    sw = selfw[:N].reshape(N, 1)

    src3b = src3.reshape(NSC * NTEC, rows32, K)
    dst3b = dst3.reshape(NSC * NTEC, rows32, K)
    norm3b = norm3.reshape(NSC * NTEC, rows32, K)

    y0p = _spmm0_kernel(x, src3b, dst3b, norm3b)
    h1a, h1b = _dense0(y0p, x, sw, W0, b0.reshape(1, 256),
                       g0.reshape(1, 256), be0.reshape(1, 256))
    y1c = _spmm1_kernel(h1a, h1b, src3b, dst3b, norm3b)
    (z,) = _dense1(y1c, h1a, h1b, sw, W1, b1.reshape(1, 256),
                   g1.reshape(1, 256), be1.reshape(1, 256), W2.reshape(1, 256))
    zpad = jnp.pad(z[:, 0], (0, NPAD - N))
    b2b = jnp.broadcast_to(b2, (L,))
    out = _spmm_z_kernel(zpad, src3, dst3, norm3, selfw, b2b)
    return out[:N].reshape(1, N)
